# baseline (speedup 1.0000x reference)
import jax
import jax.numpy as jnp
from jax.experimental import pallas as pl
from jax.experimental.pallas import tpu as pltpu

_WINDOW = 33
_HALF = _WINDOW // 2
_DIL = 4
_SEQ = 2048
_SP = _SEQ // _DIL
_D = 1024
_QD = 64


def _xsum_kernel(x_ref, o_ref):
    o_ref[0, 0, :] = jnp.sum(x_ref[0], axis=0)


def _attn_kernel(x_ref, w_ref, b_ref, xsum_ref, o_ref):
    xp = x_ref[0, 0]
    w = w_ref[...]
    bias = b_ref[0]
    qkv = jnp.dot(xp, w, preferred_element_type=jnp.float32) + bias
    q = qkv[:, :_QD]
    k = qkv[:, _QD:2 * _QD]
    v = qkv[:, 2 * _QD:]

    sumv = (
        jnp.dot(xsum_ref[0], w[:, 2 * _QD:], preferred_element_type=jnp.float32)[0]
        + _SEQ * bias[2 * _QD:]
    )

    s = jnp.dot(q, k.T, preferred_element_type=jnp.float32,
                precision=jax.lax.Precision.HIGHEST)
    ii = jax.lax.broadcasted_iota(jnp.int32, (_SP, _SP), 0)
    jj = jax.lax.broadcasted_iota(jnp.int32, (_SP, _SP), 1)
    mask = jnp.abs(ii - jj) <= _HALF
    s = jnp.where(mask, s, 0.0)
    m = jnp.max(s, axis=1, keepdims=True)
    em = jnp.exp(-m)
    p = jnp.where(mask, jnp.exp(s - m) - em, 0.0)
    numer = jnp.dot(p, v, preferred_element_type=jnp.float32) + em * sumv[None, :]
    denom = jnp.sum(p, axis=1, keepdims=True) + _SEQ * em
    o_ref[0, 0] = numer / denom


def kernel(x, WQ, bQ, WK, bK, WV, bV):
    B, S, D = x.shape
    w = jnp.concatenate([WQ, WK, WV], axis=0).T
    bias = jnp.concatenate([bQ, bK, bV])[None, :]

    xsum = pl.pallas_call(
        _xsum_kernel,
        grid=(B,),
        in_specs=[pl.BlockSpec((1, S, D), lambda b: (b, 0, 0))],
        out_specs=pl.BlockSpec((1, 1, D), lambda b: (b, 0, 0)),
        out_shape=jax.ShapeDtypeStruct((B, 1, D), jnp.float32),
    )(x)

    xt = x.reshape(B, _SP, _DIL, D).transpose(0, 2, 1, 3)
    out = pl.pallas_call(
        _attn_kernel,
        grid=(B, _DIL),
        in_specs=[
            pl.BlockSpec((1, 1, _SP, D), lambda b, p: (b, p, 0, 0)),
            pl.BlockSpec((D, 3 * _QD), lambda b, p: (0, 0)),
            pl.BlockSpec((1, 3 * _QD), lambda b, p: (0, 0)),
            pl.BlockSpec((1, 1, D), lambda b, p: (b, 0, 0)),
        ],
        out_specs=pl.BlockSpec((1, 1, _SP, _QD), lambda b, p: (b, p, 0, 0)),
        out_shape=jax.ShapeDtypeStruct((B, _DIL, _SP, _QD), jnp.float32),
        compiler_params=pltpu.CompilerParams(
            dimension_semantics=("parallel", "parallel"),
        ),
    )(xt, w, bias, xsum)

    return out.transpose(0, 2, 1, 3).reshape(B, S, _QD)


# speedup vs baseline: 1.2714x; 1.2714x over previous
import jax
import jax.numpy as jnp
from jax.experimental import pallas as pl
from jax.experimental.pallas import tpu as pltpu

_WINDOW = 33
_HALF = _WINDOW // 2
_DIL = 4
_SEQ = 2048
_SP = _SEQ // _DIL
_D = 1024
_QD = 64


def _xsum_kernel(x_ref, o_ref):
    o_ref[0, 0, :] = jnp.sum(x_ref[0], axis=0)


def _attn_kernel(x_ref, w_ref, b_ref, xsum_ref, o_ref):
    w = w_ref[...]
    bias = b_ref[0]
    sumv = (
        jnp.dot(xsum_ref[0], w[:, 2 * _QD:], preferred_element_type=jnp.float32)[0]
        + _SEQ * bias[2 * _QD:]
    )

    ii = jax.lax.broadcasted_iota(jnp.int32, (_SP, _SP), 0)
    jj = jax.lax.broadcasted_iota(jnp.int32, (_SP, _SP), 1)
    mask = jnp.abs(ii - jj) <= _HALF

    outs = []
    for h in range(2):
        xp = x_ref[0, :, h * _D:(h + 1) * _D]
        qkv = jnp.dot(xp, w, preferred_element_type=jnp.float32) + bias
        q = qkv[:, :_QD]
        k = qkv[:, _QD:2 * _QD]
        v = qkv[:, 2 * _QD:]

        s = jnp.dot(q, k.T, preferred_element_type=jnp.float32,
                    precision=jax.lax.Precision.HIGHEST)
        s = jnp.where(mask, s, 0.0)
        m = jnp.max(s, axis=1, keepdims=True)
        em = jnp.exp(-m)
        p = jnp.where(mask, jnp.exp(s - m) - em, 0.0)
        numer = jnp.dot(p, v, preferred_element_type=jnp.float32) + em * sumv[None, :]
        denom = jnp.sum(p, axis=1, keepdims=True) + _SEQ * em
        outs.append(numer / denom)

    o_ref[0] = jnp.concatenate(outs, axis=1)


def kernel(x, WQ, bQ, WK, bK, WV, bV):
    B, S, D = x.shape
    w = jnp.concatenate([WQ, WK, WV], axis=0).T
    bias = jnp.concatenate([bQ, bK, bV])[None, :]

    xsum = pl.pallas_call(
        _xsum_kernel,
        grid=(B,),
        in_specs=[pl.BlockSpec((1, S, D), lambda b: (b, 0, 0))],
        out_specs=pl.BlockSpec((1, 1, D), lambda b: (b, 0, 0)),
        out_shape=jax.ShapeDtypeStruct((B, 1, D), jnp.float32),
    )(x)

    x6 = x.reshape(B, _SP, _DIL * D)
    out = pl.pallas_call(
        _attn_kernel,
        grid=(B, _DIL // 2),
        in_specs=[
            pl.BlockSpec((1, _SP, 2 * D), lambda b, q: (b, 0, q)),
            pl.BlockSpec((D, 3 * _QD), lambda b, q: (0, 0)),
            pl.BlockSpec((1, 3 * _QD), lambda b, q: (0, 0)),
            pl.BlockSpec((1, 1, D), lambda b, q: (b, 0, 0)),
        ],
        out_specs=pl.BlockSpec((1, _SP, 2 * _QD), lambda b, q: (b, 0, q)),
        out_shape=jax.ShapeDtypeStruct((B, _SP, _DIL * _QD), jnp.float32),
        compiler_params=pltpu.CompilerParams(
            dimension_semantics=("parallel", "parallel"),
        ),
    )(x6, w, bias, xsum)

    return out.reshape(B, S, _QD)


# speedup vs baseline: 1.3287x; 1.0450x over previous
import jax
import jax.numpy as jnp
from jax.experimental import pallas as pl
from jax.experimental.pallas import tpu as pltpu

_WINDOW = 33
_HALF = _WINDOW // 2
_DIL = 4
_SEQ = 2048
_SP = _SEQ // _DIL
_D = 1024
_QD = 64


def _attn_kernel(x_ref, wqk_ref, wv_ref, b_ref, o_ref):
    wqk = wqk_ref[...]
    wv = wv_ref[...]
    bias = b_ref[0]
    bqk = bias[:2 * _QD]
    bv = bias[2 * _QD:]

    vs = []
    for p in range(_DIL):
        xp = x_ref[0, :, p * _D:(p + 1) * _D]
        vs.append(jnp.dot(xp, wv, preferred_element_type=jnp.float32) + bv)
    sumv = vs[0].sum(axis=0)
    for p in range(1, _DIL):
        sumv = sumv + vs[p].sum(axis=0)

    ii = jax.lax.broadcasted_iota(jnp.int32, (_SP, _SP), 0)
    jj = jax.lax.broadcasted_iota(jnp.int32, (_SP, _SP), 1)
    mask = jnp.abs(ii - jj) <= _HALF

    outs = []
    for p in range(_DIL):
        xp = x_ref[0, :, p * _D:(p + 1) * _D]
        qk = jnp.dot(xp, wqk, preferred_element_type=jnp.float32) + bqk
        q = qk[:, :_QD]
        k = qk[:, _QD:]
        v = vs[p]

        s = jnp.dot(q, k.T, preferred_element_type=jnp.float32,
                    precision=jax.lax.Precision.HIGHEST)
        s = jnp.where(mask, s, 0.0)
        m = jnp.max(s, axis=1, keepdims=True)
        em = jnp.exp(-m)
        pp = jnp.where(mask, jnp.exp(s - m) - em, 0.0)
        numer = jnp.dot(pp, v, preferred_element_type=jnp.float32) + em * sumv[None, :]
        denom = jnp.sum(pp, axis=1, keepdims=True) + _SEQ * em
        outs.append(numer / denom)

    o_ref[0] = jnp.concatenate(outs, axis=1)


def kernel(x, WQ, bQ, WK, bK, WV, bV):
    B, S, D = x.shape
    wqk = jnp.concatenate([WQ, WK], axis=0).T
    wv = WV.T
    bias = jnp.concatenate([bQ, bK, bV])[None, :]

    x6 = x.reshape(B, _SP, _DIL * D)
    out = pl.pallas_call(
        _attn_kernel,
        grid=(B,),
        in_specs=[
            pl.BlockSpec((1, _SP, _DIL * D), lambda b: (b, 0, 0)),
            pl.BlockSpec((D, 2 * _QD), lambda b: (0, 0)),
            pl.BlockSpec((D, _QD), lambda b: (0, 0)),
            pl.BlockSpec((1, 3 * _QD), lambda b: (0, 0)),
        ],
        out_specs=pl.BlockSpec((1, _SP, _DIL * _QD), lambda b: (b, 0, 0)),
        out_shape=jax.ShapeDtypeStruct((B, _SP, _DIL * _QD), jnp.float32),
        compiler_params=pltpu.CompilerParams(
            dimension_semantics=("arbitrary",),
        ),
    )(x6, wqk, wv, bias)

    return out.reshape(B, S, _QD)
